# baseline (speedup 1.0000x reference)
import functools

import jax
import jax.numpy as jnp
import numpy as np
from jax import lax
from jax.experimental import pallas as pl
from jax.experimental.pallas import tpu as pltpu
from jax.experimental.pallas import tpu_sc as plsc

_PRIMES = (4993, 4999, 5003, 5009, 5011)
_ALPHA = 0.0001
_N = 16384
_NC, _NS, _L = 2, 16, 16
_NW = _NC * _NS
_CHUNK = _N // _NW
_PAD = 5024
_LN2 = float(np.log(2.0))
_LOG2_COEF = (1.4426943455113115, -0.7212949323750789, 0.4799159780292521,
              -0.35278695884196, 0.2547762137791782, -0.1540769304318734,
              0.06298603700359981, -0.012214252057937003)

_mesh = plsc.VectorSubcoreMesh(core_axis_name="c", subcore_axis_name="s")


def _body(items_hbm, a0h, a1h, a2h, a3h, a4h, b0h, b1h, b2h, b3h, b4h,
          t_hbm, out_hbm, items_v, a_tabs, b_tabs, t_v, out_v, sems):
    wid = lax.axis_index("s") * _NC + lax.axis_index("c")
    base = wid * _CHUNK
    c_items = pltpu.async_copy(
        items_hbm.at[pl.ds(base, _CHUNK)], items_v, sems[5])
    c_t = pltpu.async_copy(t_hbm, t_v, sems[5])
    tab_copies = []
    for k, (ah, bh) in enumerate(zip((a0h, a1h, a2h, a3h, a4h),
                                     (b0h, b1h, b2h, b3h, b4h))):
        tab_copies.append((pltpu.async_copy(ah, a_tabs[k], sems[k]),
                           pltpu.async_copy(bh, b_tabs[k], sems[k])))
    c_items.wait()
    c_t.wait()
    t1 = plsc.load_gather(t_v, [jnp.zeros((_L,), jnp.int32)]) + 1.0
    nv = _CHUNK // _L

    for k, p in enumerate(_PRIMES):
        ca, cb = tab_copies[k]
        ca.wait()
        cb.wait()

        def pass_k(i, carry, k=k, p=p):
            off = pl.multiple_of(i * _L, _L)
            it = items_v[pl.ds(off, _L)]
            itf = it.astype(jnp.float32)
            q = (itf * np.float32(1.0 / p)).astype(jnp.int32)
            r = it - q * p
            r = jnp.where(r < 0, r + p, r)
            r = jnp.where(r >= p, r - p, r)
            ag = plsc.load_gather(a_tabs[k], [r])
            bg = plsc.load_gather(b_tabs[k], [r])
            delta = (1.0 - _ALPHA) * bg + _ALPHA * (t1 - ag)
            if k:
                delta = jnp.maximum(delta, out_v[pl.ds(off, _L)])
            out_v[pl.ds(off, _L)] = delta
            return carry

        lax.fori_loop(0, nv, pass_k, 0)

    def log_pass(i, carry):
        off = pl.multiple_of(i * _L, _L)
        m = out_v[pl.ds(off, _L)]
        yi = lax.bitcast_convert_type(m, jnp.int32)
        e = lax.shift_right_arithmetic(yi, 23) - 127
        mant = lax.bitcast_convert_type(
            (yi & 0x007FFFFF) | 0x3F800000, jnp.float32)
        s = mant - 1.0
        acc = jnp.float32(_LOG2_COEF[-1])
        for cc in _LOG2_COEF[-2::-1]:
            acc = acc * s + jnp.float32(cc)
        log2m = e.astype(jnp.float32) + s * acc
        out_v[pl.ds(off, _L)] = np.float32(-_LN2) * log2m
        return carry

    lax.fori_loop(0, nv, log_pass, 0)
    pltpu.sync_copy(out_v, out_hbm.at[pl.ds(base, _CHUNK)])


_sc_call = functools.partial(
    pl.kernel,
    out_type=jax.ShapeDtypeStruct((_N,), jnp.float32),
    mesh=_mesh,
    compiler_params=pltpu.CompilerParams(needs_layout_passes=False),
    scratch_types=[
        pltpu.VMEM((_CHUNK,), jnp.int32),
        [pltpu.VMEM((p,), jnp.float32) for p in _PRIMES],
        [pltpu.VMEM((p,), jnp.float32) for p in _PRIMES],
        pltpu.VMEM((1,), jnp.float32),
        pltpu.VMEM((_CHUNK,), jnp.float32),
        [pltpu.SemaphoreType.DMA for _ in range(6)],
    ],
)(_body)


def kernel(items, A0, A1, A2, A3, A4, B0, B1, B2, B3, B4, t):
    return _sc_call(items, A0, A1, A2, A3, A4, B0, B1, B2, B3, B4, t)


# speedup vs baseline: 1.0504x; 1.0504x over previous
import functools

import jax
import jax.numpy as jnp
import numpy as np
from jax import lax
from jax.experimental import pallas as pl
from jax.experimental.pallas import tpu as pltpu
from jax.experimental.pallas import tpu_sc as plsc

_PRIMES = (4993, 4999, 5003, 5009, 5011)
_ALPHA = 0.0001
_N = 16384
_NC, _NS, _L = 2, 16, 16
_NW = _NC * _NS
_CHUNK = _N // _NW
_PAD = 5024
_LN2 = float(np.log(2.0))
_LOG2_COEF = (1.4426943455113115, -0.7212949323750789, 0.4799159780292521,
              -0.35278695884196, 0.2547762137791782, -0.1540769304318734,
              0.06298603700359981, -0.012214252057937003)

_mesh = plsc.VectorSubcoreMesh(core_axis_name="c", subcore_axis_name="s")


def _body(items_hbm, a0h, a1h, a2h, a3h, a4h, b0h, b1h, b2h, b3h, b4h,
          t_hbm, out_hbm, items_v, a_tabs, b_tabs, t_v, out_v, sems):
    wid = lax.axis_index("s") * _NC + lax.axis_index("c")
    base = wid * _CHUNK
    c_items = pltpu.async_copy(
        items_hbm.at[pl.ds(base, _CHUNK)], items_v, sems[5])
    c_t = pltpu.async_copy(t_hbm, t_v, sems[5])
    tab_copies = []
    for k, (ah, bh) in enumerate(zip((a0h, a1h, a2h, a3h, a4h),
                                     (b0h, b1h, b2h, b3h, b4h))):
        tab_copies.append((pltpu.async_copy(ah, a_tabs[k], sems[k]),
                           pltpu.async_copy(bh, b_tabs[k], sems[k])))
    c_items.wait()
    c_t.wait()
    for ca, cb in tab_copies:
        ca.wait()
        cb.wait()
    t1 = plsc.load_gather(t_v, [jnp.zeros((_L,), jnp.int32)]) + 1.0

    def step(i, carry):
        off = pl.multiple_of(i * _L, _L)
        it = items_v[pl.ds(off, _L)]
        itf = it.astype(jnp.float32)
        m = None
        for k, p in enumerate(_PRIMES):
            q = (itf * np.float32(1.0 / p)).astype(jnp.int32)
            r = it - q * p
            r = jnp.where(r < 0, r + p, r)
            r = jnp.where(r >= p, r - p, r)
            ag = plsc.load_gather(a_tabs[k], [r])
            bg = plsc.load_gather(b_tabs[k], [r])
            delta = (1.0 - _ALPHA) * bg + _ALPHA * (t1 - ag)
            m = delta if m is None else jnp.maximum(m, delta)
        yi = lax.bitcast_convert_type(m, jnp.int32)
        e = lax.shift_right_arithmetic(yi, 23) - 127
        mant = lax.bitcast_convert_type(
            (yi & 0x007FFFFF) | 0x3F800000, jnp.float32)
        s = mant - 1.0
        acc = jnp.float32(_LOG2_COEF[-1])
        for cc in _LOG2_COEF[-2::-1]:
            acc = acc * s + jnp.float32(cc)
        log2m = e.astype(jnp.float32) + s * acc
        out_v[pl.ds(off, _L)] = np.float32(-_LN2) * log2m
        return carry

    lax.fori_loop(0, _CHUNK // _L, step, 0)
    pltpu.sync_copy(out_v, out_hbm.at[pl.ds(base, _CHUNK)])


_sc_call = functools.partial(
    pl.kernel,
    out_type=jax.ShapeDtypeStruct((_N,), jnp.float32),
    mesh=_mesh,
    compiler_params=pltpu.CompilerParams(needs_layout_passes=False),
    scratch_types=[
        pltpu.VMEM((_CHUNK,), jnp.int32),
        [pltpu.VMEM((p,), jnp.float32) for p in _PRIMES],
        [pltpu.VMEM((p,), jnp.float32) for p in _PRIMES],
        pltpu.VMEM((1,), jnp.float32),
        pltpu.VMEM((_CHUNK,), jnp.float32),
        [pltpu.SemaphoreType.DMA for _ in range(6)],
    ],
)(_body)


def kernel(items, A0, A1, A2, A3, A4, B0, B1, B2, B3, B4, t):
    return _sc_call(items, A0, A1, A2, A3, A4, B0, B1, B2, B3, B4, t)
